# Initial kernel scaffold; baseline (speedup 1.0000x reference)
import jax
import jax.numpy as jnp
from jax.experimental import pallas as pl


def kernel(event_type_id, src_id, src_mask, dst_id, dst_mask, label_id, label_mask, node_embeddings, label_embeddings):
    raise NotImplementedError("write your pallas kernel here")


# speedup vs baseline: 2.9190x; 2.9190x over previous
import functools

import jax
import jax.numpy as jnp
from jax import lax
from jax.experimental import pallas as pl
from jax.experimental.pallas import tpu as pltpu
from jax.experimental.pallas import tpu_sc as plsc

B, S, H = 1024, 200, 64
BS = B * S
NC, NS = 2, 16
NW = NC * NS
ROWS_PER_W = BS // NW
CHUNK = 256
NCHUNK = ROWS_PER_W // CHUNK
G = 128


def _body(sid_hbm, did_hbm, lid_hbm, et_hbm, sm_hbm, dm_hbm, lm_hbm,
          node_hbm, label_hbm, out_hbm,
          sidx, didx, lidx, etv, smv, dmv, lmv,
          srows, drows, lrows, etblk, sem_in, sem_g, sem_out):
    wid = lax.axis_index("s") * NC + lax.axis_index("c")

    def chunk_body(c, _):
        base = (wid * NCHUNK + c) * CHUNK
        row0 = (wid * NCHUNK + c) * (CHUNK // G)

        ins = [
            pltpu.async_copy(sid_hbm.at[pl.ds(row0, CHUNK // G)], sidx, sem_in),
            pltpu.async_copy(did_hbm.at[pl.ds(row0, CHUNK // G)], didx, sem_in),
            pltpu.async_copy(lid_hbm.at[pl.ds(row0, CHUNK // G)], lidx, sem_in),
            pltpu.async_copy(et_hbm.at[pl.ds(base, CHUNK)], etv, sem_in),
            pltpu.async_copy(sm_hbm.at[pl.ds(base, CHUNK)], smv, sem_in),
            pltpu.async_copy(dm_hbm.at[pl.ds(base, CHUNK)], dmv, sem_in),
            pltpu.async_copy(lm_hbm.at[pl.ds(base, CHUNK)], lmv, sem_in),
        ]
        for d in ins:
            d.wait()

        gs = []
        for j in range(CHUNK // G):
            dst = pl.ds(j * G, G)
            gs.append(pltpu.async_copy(node_hbm.at[sidx.at[j]],
                                       srows.at[dst], sem_g))
            gs.append(pltpu.async_copy(node_hbm.at[didx.at[j]],
                                       drows.at[dst], sem_g))
            gs.append(pltpu.async_copy(label_hbm.at[lidx.at[j]],
                                       lrows.at[dst], sem_g))
        for d in gs:
            d.wait()

        def row_body(r, _):
            ridx = jnp.full((16,), r, jnp.int32)
            et = plsc.load_gather(etv, [ridx])
            sm = plsc.load_gather(smv, [ridx])
            dm = plsc.load_gather(dmv, [ridx])
            lm = plsc.load_gather(lmv, [ridx])
            for q in range(H // 16):
                sl = pl.ds(q * 16, 16)
                etblk[r, sl] = et
                srows[r, sl] = srows[r, sl] * sm
                drows[r, sl] = drows[r, sl] * dm
                lrows[r, sl] = lrows[r, sl] * lm
            return _

        lax.fori_loop(0, CHUNK, row_body, None)

        rows = pl.ds(base, CHUNK)
        outs = [
            pltpu.async_copy(etblk, out_hbm.at[rows, pl.ds(0 * H, H)], sem_out),
            pltpu.async_copy(srows, out_hbm.at[rows, pl.ds(1 * H, H)], sem_out),
            pltpu.async_copy(drows, out_hbm.at[rows, pl.ds(2 * H, H)], sem_out),
            pltpu.async_copy(lrows, out_hbm.at[rows, pl.ds(3 * H, H)], sem_out),
        ]
        for d in outs:
            d.wait()
        return _

    lax.fori_loop(0, NCHUNK, chunk_body, None)


@jax.jit
def _encode(sid, did, lid, et, sm, dm, lm, node_emb, label_emb):
    mesh = plsc.VectorSubcoreMesh(core_axis_name="c", subcore_axis_name="s")
    f = functools.partial(
        pl.kernel,
        out_type=jax.ShapeDtypeStruct((BS, 4 * H), jnp.float32),
        mesh=mesh,
        compiler_params=pltpu.CompilerParams(use_tc_tiling_on_sc=False,
                                             needs_layout_passes=False),
        scratch_types=[
            pltpu.VMEM((CHUNK // G, G), jnp.int32),
            pltpu.VMEM((CHUNK // G, G), jnp.int32),
            pltpu.VMEM((CHUNK // G, G), jnp.int32),
            pltpu.VMEM((CHUNK,), jnp.float32),
            pltpu.VMEM((CHUNK,), jnp.float32),
            pltpu.VMEM((CHUNK,), jnp.float32),
            pltpu.VMEM((CHUNK,), jnp.float32),
            pltpu.VMEM((CHUNK, H), jnp.float32),
            pltpu.VMEM((CHUNK, H), jnp.float32),
            pltpu.VMEM((CHUNK, H), jnp.float32),
            pltpu.VMEM((CHUNK, H), jnp.float32),
            pltpu.SemaphoreType.DMA,
            pltpu.SemaphoreType.DMA,
            pltpu.SemaphoreType.DMA,
        ],
    )(_body)
    return f(sid, did, lid, et, sm, dm, lm, node_emb, label_emb)


def kernel(event_type_id, src_id, src_mask, dst_id, dst_mask, label_id,
           label_mask, node_embeddings, label_embeddings):
    sid = src_id.astype(jnp.int32).reshape(BS // G, G)
    did = dst_id.astype(jnp.int32).reshape(BS // G, G)
    lid = label_id.astype(jnp.int32).reshape(BS // G, G)
    et = event_type_id.reshape(BS)
    sm = src_mask.reshape(BS)
    dm = dst_mask.reshape(BS)
    lm = label_mask.reshape(BS)
    out = _encode(sid, did, lid, et, sm, dm, lm,
                  node_embeddings, label_embeddings)
    return out.reshape(B, S, 4 * H)


# speedup vs baseline: 3.3921x; 1.1621x over previous
import functools

import jax
import jax.numpy as jnp
from jax import lax
from jax.experimental import pallas as pl
from jax.experimental.pallas import tpu as pltpu
from jax.experimental.pallas import tpu_sc as plsc

B, S, H = 1024, 200, 64
BS = B * S
NC, NS = 2, 16
NW = NC * NS
ROWS_PER_W = BS // NW
CHUNK = 128
NCHUNK = ROWS_PER_W // CHUNK
NBUF = 3


def _body(ids_hbm, scal_hbm, node_hbm, label_hbm, out_hbm, *scratch):
    ids_v = scratch[0:3]
    scal_v = scratch[3:6]
    rows_v = [scratch[6 + 4 * b:6 + 4 * b + 4] for b in range(3)]
    sem_in = scratch[18:21]
    sem_g = scratch[21:24]
    sem_out = scratch[24:27]

    wid = lax.axis_index("s") * NC + lax.axis_index("c")

    def in_descs(c, b):
        cg = wid * NCHUNK + c
        return [
            pltpu.make_async_copy(ids_hbm.at[:, pl.ds(cg, 1), :], ids_v[b],
                                  sem_in[b]),
            pltpu.make_async_copy(scal_hbm.at[:, pl.ds(cg * CHUNK, CHUNK)],
                                  scal_v[b], sem_in[b]),
        ]

    def g_descs(c, b):
        return [
            pltpu.make_async_copy(node_hbm.at[ids_v[b].at[0, 0]],
                                  rows_v[b][1], sem_g[b]),
            pltpu.make_async_copy(node_hbm.at[ids_v[b].at[1, 0]],
                                  rows_v[b][2], sem_g[b]),
            pltpu.make_async_copy(label_hbm.at[ids_v[b].at[2, 0]],
                                  rows_v[b][3], sem_g[b]),
        ]

    def out_descs(c, b):
        rows = pl.ds((wid * NCHUNK + c) * CHUNK, CHUNK)
        return [
            pltpu.make_async_copy(rows_v[b][q],
                                  out_hbm.at[rows, pl.ds(q * H, H)],
                                  sem_out[b])
            for q in range(4)
        ]

    def fire(descs):
        for d in descs:
            d.start()

    def drain(descs):
        for d in descs:
            d.wait()

    def compute(b):
        etblk, srows, drows, lrows = rows_v[b]
        scal = scal_v[b]
        i0 = jnp.full((16,), 0, jnp.int32)
        i1 = jnp.full((16,), 1, jnp.int32)
        i2 = jnp.full((16,), 2, jnp.int32)
        i3 = jnp.full((16,), 3, jnp.int32)

        def row_body(r, _):
            ridx = jnp.full((16,), r, jnp.int32)
            et = plsc.load_gather(scal, [i0, ridx])
            sm = plsc.load_gather(scal, [i1, ridx])
            dm = plsc.load_gather(scal, [i2, ridx])
            lm = plsc.load_gather(scal, [i3, ridx])
            for q in range(H // 16):
                sl = pl.ds(q * 16, 16)
                etblk[r, sl] = et
                srows[r, sl] = srows[r, sl] * sm
                drows[r, sl] = drows[r, sl] * dm
                lrows[r, sl] = lrows[r, sl] * lm
            return _

        lax.fori_loop(0, CHUNK, row_body, None)

    def iter_ops(c, b, *, first_out_wait=True, fire_next_g=True,
                 fire_next_in=True):
        drain(g_descs(c, b))
        if fire_next_g:
            bn = (b + 1) % NBUF
            drain(in_descs(c + 1, bn))
            if first_out_wait:
                drain(out_descs(c - 2, bn))
            fire(g_descs(c + 1, bn))
        compute(b)
        fire(out_descs(c, b))
        if fire_next_in:
            fire(in_descs(c + 2, (b + 2) % NBUF))

    fire(in_descs(0, 0))
    fire(in_descs(1, 1))
    drain(in_descs(0, 0))
    fire(g_descs(0, 0))

    iter_ops(0, 0, first_out_wait=False)
    iter_ops(1, 1, first_out_wait=False)
    iter_ops(2, 2)

    n_steady = NCHUNK - 5
    def outer(cc, _):
        for j in range(NBUF):
            iter_ops(3 + cc * NBUF + j, j)
        return _

    lax.fori_loop(0, n_steady // NBUF, outer, None)

    iter_ops(NCHUNK - 2, (NCHUNK - 2) % NBUF, fire_next_in=False)
    c = NCHUNK - 1
    b = c % NBUF
    drain(g_descs(c, b))
    drain(out_descs(c - 2, (b + 1) % NBUF))
    compute(b)
    fire(out_descs(c, b))
    drain(out_descs(NCHUNK - 2, (NCHUNK - 2) % NBUF))
    drain(out_descs(NCHUNK - 1, b))


@jax.jit
def _encode(ids, scal, node_emb, label_emb):
    mesh = plsc.VectorSubcoreMesh(core_axis_name="c", subcore_axis_name="s")
    scratch = (
        [pltpu.VMEM((3, 1, CHUNK), jnp.int32) for _ in range(NBUF)]
        + [pltpu.VMEM((4, CHUNK), jnp.float32) for _ in range(NBUF)]
        + [pltpu.VMEM((CHUNK, H), jnp.float32) for _ in range(4 * NBUF)]
        + [pltpu.SemaphoreType.DMA for _ in range(3 * NBUF)]
    )
    f = functools.partial(
        pl.kernel,
        out_type=jax.ShapeDtypeStruct((BS, 4 * H), jnp.float32),
        mesh=mesh,
        compiler_params=pltpu.CompilerParams(use_tc_tiling_on_sc=False,
                                             needs_layout_passes=False),
        scratch_types=scratch,
    )(_body)
    return f(ids, scal, node_emb, label_emb)


def kernel(event_type_id, src_id, src_mask, dst_id, dst_mask, label_id,
           label_mask, node_embeddings, label_embeddings):
    ids = jnp.stack([src_id.astype(jnp.int32).reshape(BS),
                     dst_id.astype(jnp.int32).reshape(BS),
                     label_id.astype(jnp.int32).reshape(BS)]
                    ).reshape(3, BS // CHUNK, CHUNK)
    scal = jnp.stack([event_type_id.reshape(BS),
                      src_mask.reshape(BS),
                      dst_mask.reshape(BS),
                      label_mask.reshape(BS)])
    out = _encode(ids, scal, node_embeddings, label_embeddings)
    return out.reshape(B, S, 4 * H)


# speedup vs baseline: 5.4501x; 1.6067x over previous
import functools

import jax
import jax.numpy as jnp
from jax import lax
from jax.experimental import pallas as pl
from jax.experimental.pallas import tpu as pltpu
from jax.experimental.pallas import tpu_sc as plsc

B, S, H = 1024, 200, 64
BS = B * S
NC, NS = 2, 16
NW = NC * NS
ROWS_PER_W = BS // NW
CHUNK = 128
NCHUNK = ROWS_PER_W // CHUNK
NBUF = 3


def _body(ids_hbm, scal_hbm, node_hbm, label_hbm, out_hbm, *scratch):
    ids_v = scratch[0:3]
    scal_v = scratch[3:6]
    rows_v = [scratch[6 + 4 * b:6 + 4 * b + 4] for b in range(3)]
    sem_in = scratch[18:21]
    sem_g = scratch[21:24]
    sem_out = scratch[24:27]

    wid = lax.axis_index("s") * NC + lax.axis_index("c")

    def in_descs(c, b):
        cg = wid * NCHUNK + c
        return [
            pltpu.make_async_copy(ids_hbm.at[:, pl.ds(cg, 1), :], ids_v[b],
                                  sem_in[b]),
            pltpu.make_async_copy(scal_hbm.at[:, pl.ds(cg * CHUNK, CHUNK)],
                                  scal_v[b], sem_in[b]),
        ]

    def g_descs(c, b):
        return [
            pltpu.make_async_copy(node_hbm.at[ids_v[b].at[0, 0]],
                                  rows_v[b][1], sem_g[b]),
            pltpu.make_async_copy(node_hbm.at[ids_v[b].at[1, 0]],
                                  rows_v[b][2], sem_g[b]),
            pltpu.make_async_copy(label_hbm.at[ids_v[b].at[2, 0]],
                                  rows_v[b][3], sem_g[b]),
        ]

    def out_descs(c, b):
        rows = pl.ds((wid * NCHUNK + c) * CHUNK, CHUNK)
        return [
            pltpu.make_async_copy(rows_v[b][q],
                                  out_hbm.at[rows, pl.ds(q * H, H)],
                                  sem_out[b])
            for q in range(4)
        ]

    def fire(descs):
        for d in descs:
            d.start()

    def drain(descs):
        for d in descs:
            d.wait()

    def compute(b):
        etblk, srows, drows, lrows = rows_v[b]
        scal = scal_v[b]
        i0 = jnp.full((16,), 0, jnp.int32)
        i1 = jnp.full((16,), 1, jnp.int32)
        i2 = jnp.full((16,), 2, jnp.int32)
        i3 = jnp.full((16,), 3, jnp.int32)

        def row_body(r, _):
            ridx = jnp.full((16,), r, jnp.int32)
            et = plsc.load_gather(scal, [i0, ridx])
            sm = plsc.load_gather(scal, [i1, ridx])
            dm = plsc.load_gather(scal, [i2, ridx])
            lm = plsc.load_gather(scal, [i3, ridx])
            for q in range(H // 16):
                sl = pl.ds(q * 16, 16)
                etblk[r, sl] = et
                srows[r, sl] = srows[r, sl] * sm
                drows[r, sl] = drows[r, sl] * dm
                lrows[r, sl] = lrows[r, sl] * lm
            return _

        lax.fori_loop(0, CHUNK, row_body, None, unroll=8)

    def iter_ops(c, b, *, first_out_wait=True, fire_next_g=True,
                 fire_next_in=True):
        drain(g_descs(c, b))
        if fire_next_g:
            bn = (b + 1) % NBUF
            drain(in_descs(c + 1, bn))
            if first_out_wait:
                drain(out_descs(c - 2, bn))
            fire(g_descs(c + 1, bn))
        compute(b)
        fire(out_descs(c, b))
        if fire_next_in:
            fire(in_descs(c + 2, (b + 2) % NBUF))

    fire(in_descs(0, 0))
    fire(in_descs(1, 1))
    drain(in_descs(0, 0))
    fire(g_descs(0, 0))

    iter_ops(0, 0, first_out_wait=False)
    iter_ops(1, 1, first_out_wait=False)
    iter_ops(2, 2)

    n_steady = NCHUNK - 5
    def outer(cc, _):
        for j in range(NBUF):
            iter_ops(3 + cc * NBUF + j, j)
        return _

    lax.fori_loop(0, n_steady // NBUF, outer, None)

    iter_ops(NCHUNK - 2, (NCHUNK - 2) % NBUF, fire_next_in=False)
    c = NCHUNK - 1
    b = c % NBUF
    drain(g_descs(c, b))
    drain(out_descs(c - 2, (b + 1) % NBUF))
    compute(b)
    fire(out_descs(c, b))
    drain(out_descs(NCHUNK - 2, (NCHUNK - 2) % NBUF))
    drain(out_descs(NCHUNK - 1, b))


@jax.jit
def _encode(ids, scal, node_emb, label_emb):
    mesh = plsc.VectorSubcoreMesh(core_axis_name="c", subcore_axis_name="s")
    scratch = (
        [pltpu.VMEM((3, 1, CHUNK), jnp.int32) for _ in range(NBUF)]
        + [pltpu.VMEM((4, CHUNK), jnp.float32) for _ in range(NBUF)]
        + [pltpu.VMEM((CHUNK, H), jnp.float32) for _ in range(4 * NBUF)]
        + [pltpu.SemaphoreType.DMA for _ in range(3 * NBUF)]
    )
    f = functools.partial(
        pl.kernel,
        out_type=jax.ShapeDtypeStruct((BS, 4 * H), jnp.float32),
        mesh=mesh,
        compiler_params=pltpu.CompilerParams(use_tc_tiling_on_sc=False,
                                             needs_layout_passes=False),
        scratch_types=scratch,
    )(_body)
    return f(ids, scal, node_emb, label_emb)


def kernel(event_type_id, src_id, src_mask, dst_id, dst_mask, label_id,
           label_mask, node_embeddings, label_embeddings):
    ids = jnp.stack([src_id.astype(jnp.int32).reshape(BS),
                     dst_id.astype(jnp.int32).reshape(BS),
                     label_id.astype(jnp.int32).reshape(BS)]
                    ).reshape(3, BS // CHUNK, CHUNK)
    scal = jnp.stack([event_type_id.reshape(BS),
                      src_mask.reshape(BS),
                      dst_mask.reshape(BS),
                      label_mask.reshape(BS)])
    out = _encode(ids, scal, node_embeddings, label_embeddings)
    return out.reshape(B, S, 4 * H)
